# baseline (speedup 1.0000x reference)
import functools

import jax
import jax.numpy as jnp
import numpy as np
from jax import lax
from jax.experimental import pallas as pl
from jax.experimental.pallas import tpu as pltpu
from jax.experimental.pallas import tpu_sc as plsc

NUM_EMB = 1000000
DIM = 64
QMAX = 127.0
BATCH = 16384
HIST = 50

_NW = 32
_BPW = BATCH // _NW
_RB = 256
_NROUND = HIST * 2

_SROWS = NUM_EMB // 16
_SCH = 244
_STAIL = _SROWS - _SCH * _RB

_SIGN_MASK = np.uint32(0x80000000)
_MAGIC_BITS = np.uint32(0x4B000000)


def _gather_fq(table, idx_flat):
    mesh = plsc.VectorSubcoreMesh(core_axis_name="c", subcore_axis_name="s")

    @functools.partial(
        pl.kernel,
        mesh=mesh,
        out_type=jax.ShapeDtypeStruct((HIST, DIM, BATCH), jnp.float32),
        scratch_types=[
            pltpu.VMEM((_BPW * HIST,), jnp.int32),
            [pltpu.VMEM((_RB,), jnp.int32) for _ in range(2)],
            [pltpu.VMEM((_RB, DIM), jnp.float32) for _ in range(2)],
            [pltpu.VMEM((DIM, _RB), jnp.float32) for _ in range(2)],
            pltpu.VMEM((16,), jnp.float32),
            pltpu.VMEM((16, 16), jnp.float32),
            pltpu.VMEM_SHARED((16, 16), jnp.float32),
            [pltpu.SemaphoreType.DMA for _ in range(2)],
            [pltpu.SemaphoreType.DMA for _ in range(2)],
        ],
        compiler_params=pltpu.CompilerParams(use_tc_tiling_on_sc=False,
                                             needs_layout_passes=False),
    )
    def k(table_hbm, idx_hbm, out_hbm, slab_v, glist, rin, tbuf, red_v,
          redall_v, shared, sem_g, sem_o):
        cid = lax.axis_index("c")
        sid = lax.axis_index("s")
        wid = sid * 2 + cid
        wb = wid * _BPW

        srow = sid * _SROWS

        def scan_rows(b, nrows, accs):
            def row_body(r2, accs):
                new = []
                for j in range(2):
                    for c in range(DIM // 16):
                        a = accs[j * 4 + c]
                        v = rin[b][2 * r2 + j, pl.ds(c * 16, 16)]
                        new.append(jnp.maximum(a, jnp.abs(v)))
                return tuple(new)

            return lax.fori_loop(0, nrows // 2, row_body, accs,
                                 unroll=False)

        for b in range(2):
            pltpu.async_copy(table_hbm.at[pl.ds(srow + b * _RB, _RB)],
                             rin[b], sem_g[b])

        accs = tuple(jnp.zeros((16,), jnp.float32) for _ in range(8))

        def scan_pair(p, accs):
            for b in range(2):
                j = 2 * p + b
                pltpu.make_async_copy(
                    table_hbm.at[pl.ds(srow, _RB)], rin[b], sem_g[b]).wait()
                accs = scan_rows(b, _RB, accs)

                @pl.when(p < _SCH // 2 - 1)
                def _prefetch():
                    pltpu.async_copy(
                        table_hbm.at[pl.ds(srow + (j + 2) * _RB, _RB)],
                        rin[b], sem_g[b])

            return accs

        accs = lax.fori_loop(0, _SCH // 2, scan_pair, accs, unroll=False)

        pltpu.sync_copy(table_hbm.at[pl.ds(srow + _SCH * _RB, _STAIL)],
                        rin[0].at[pl.ds(0, _STAIL)])
        accs = scan_rows(0, _STAIL, accs)
        m = accs[0]
        for a in accs[1:]:
            m = jnp.maximum(m, a)

        red_v[...] = m
        pltpu.sync_copy(red_v, shared.at[sid])
        plsc.subcore_barrier()
        pltpu.sync_copy(shared, redall_v)
        for t in range(16):
            m = jnp.maximum(m, redall_v[t, :])
        absmax_v = jnp.full((16,), jnp.max(m), jnp.float32)
        s = jnp.maximum(absmax_v / QMAX, 1e-8)
        rs = 1.0 / s

        pltpu.sync_copy(idx_hbm.at[pl.ds(wb * HIST, _BPW * HIST)], slab_v)

        lanes = jax.lax.iota(jnp.int32, 16)

        def build_glist(q, r):
            h = r >> 1
            half = r & 1

            def blk(kk, _):
                sidx = (half * _RB + kk * 16 + lanes) * HIST + h
                vals = plsc.load_gather(slab_v, [sidx])
                glist[q][pl.ds(kk * 16, 16)] = vals
                return 0

            lax.fori_loop(0, _RB // 16, blk, 0, unroll=False)

        def start_gather(q, r):
            build_glist(q, r)
            pltpu.async_copy(table_hbm.at[glist[q]], rin[q], sem_g[q])

        rowidx = [c * 16 + lanes for c in range(DIM // 16)]

        def dequant_t(q):
            def row_body(i, _):
                ci = jnp.full((16,), i, jnp.int32)
                for c in range(DIM // 16):
                    v = rin[q][i, pl.ds(c * 16, 16)]
                    t = v * rs
                    tb = plsc.bitcast(t, jnp.uint32)
                    csign = plsc.bitcast((tb & _SIGN_MASK) | _MAGIC_BITS,
                                         jnp.float32)
                    qv = ((t + csign) - csign) * s
                    plsc.store_scatter(tbuf[q], [rowidx[c], ci], qv)
                return 0

            lax.fori_loop(0, _RB, row_body, 0, unroll=False)

        for q in range(2):
            start_gather(q, q)

        def round_body(p, _):
            for q in range(2):
                r = 2 * p + q
                h = r >> 1
                half = r & 1
                pltpu.make_async_copy(table_hbm.at[glist[q]], rin[q],
                                      sem_g[q]).wait()

                @pl.when(p >= 1)
                def _wait_out():
                    pltpu.make_async_copy(
                        tbuf[q], out_hbm.at[0, :, pl.ds(0, _RB)],
                        sem_o[q]).wait()

                dequant_t(q)
                pltpu.async_copy(
                    tbuf[q],
                    out_hbm.at[h, :, pl.ds(wb + half * _RB, _RB)],
                    sem_o[q])

                @pl.when(p < _NROUND // 2 - 1)
                def _prefetch():
                    start_gather(q, r + 2)

            return 0

        lax.fori_loop(0, _NROUND // 2, round_body, 0, unroll=False)

        for q in range(2):
            pltpu.make_async_copy(tbuf[q], out_hbm.at[0, :, pl.ds(0, _RB)],
                                  sem_o[q]).wait()

    return k(table, idx_flat)


def kernel(x, weight):
    out = _gather_fq(weight, x.reshape(-1))
    return jnp.transpose(out, (2, 0, 1))


# speedup vs baseline: 1.7119x; 1.7119x over previous
import functools

import jax
import jax.numpy as jnp
import numpy as np
from jax import lax
from jax.experimental import pallas as pl
from jax.experimental.pallas import tpu as pltpu
from jax.experimental.pallas import tpu_sc as plsc

NUM_EMB = 1000000
DIM = 64
QMAX = 127.0
BATCH = 16384
HIST = 50

_B = BATCH * HIST
_NW = 32
_B_PER_W = _B // _NW
_CHUNK = 400
_NCHUNK = _B_PER_W // _CHUNK

_SROWS = NUM_EMB // 16
_SCH = 156
_STAIL = _SROWS - _SCH * _CHUNK

_SIGN_MASK = np.uint32(0x80000000)
_MAGIC_BITS = np.uint32(0x4B000000)


def _gather_fq(table, idx_flat):
    mesh = plsc.VectorSubcoreMesh(core_axis_name="c", subcore_axis_name="s")

    @functools.partial(
        pl.kernel,
        mesh=mesh,
        out_type=jax.ShapeDtypeStruct((_B // 2, 2 * DIM), jnp.float32),
        scratch_types=[
            pltpu.VMEM((2, _CHUNK), jnp.int32),
            [pltpu.VMEM((_CHUNK, DIM), jnp.float32) for _ in range(2)],
            [pltpu.VMEM((_CHUNK // 2, 2 * DIM), jnp.float32)
             for _ in range(2)],
            pltpu.VMEM((16,), jnp.float32),
            pltpu.VMEM((16, 16), jnp.float32),
            pltpu.VMEM_SHARED((16, 16), jnp.float32),
            [pltpu.SemaphoreType.DMA for _ in range(2)],
            [pltpu.SemaphoreType.DMA for _ in range(2)],
        ],
        compiler_params=pltpu.CompilerParams(use_tc_tiling_on_sc=False,
                                             needs_layout_passes=False),
    )
    def k(table_hbm, idx_hbm, out_hbm, idx_v, rin, rout, red_v, redall_v,
          shared, sem_g, sem_o):
        cid = lax.axis_index("c")
        sid = lax.axis_index("s")
        wid = sid * 2 + cid
        base = wid * _B_PER_W
        base2 = wid * (_B_PER_W // 2)

        srow = sid * _SROWS

        def scan_rows(b, nrows, accs):
            def row_body(r2, accs):
                new = []
                for j in range(2):
                    for c in range(DIM // 16):
                        a = accs[j * 4 + c]
                        v = rin[b][2 * r2 + j, pl.ds(c * 16, 16)]
                        new.append(jnp.maximum(a, jnp.abs(v)))
                return tuple(new)

            return lax.fori_loop(0, nrows // 2, row_body, accs,
                                 unroll=False)

        for b in range(2):
            pltpu.async_copy(table_hbm.at[pl.ds(srow + b * _CHUNK, _CHUNK)],
                             rin[b], sem_g[b])

        accs = tuple(jnp.zeros((16,), jnp.float32) for _ in range(8))

        def scan_pair(p, accs):
            for b in range(2):
                j = 2 * p + b
                pltpu.make_async_copy(
                    table_hbm.at[pl.ds(srow, _CHUNK)], rin[b],
                    sem_g[b]).wait()
                accs = scan_rows(b, _CHUNK, accs)

                @pl.when(p < _SCH // 2 - 1)
                def _prefetch():
                    pltpu.async_copy(
                        table_hbm.at[pl.ds(srow + (j + 2) * _CHUNK, _CHUNK)],
                        rin[b], sem_g[b])

            return accs

        accs = lax.fori_loop(0, _SCH // 2, scan_pair, accs, unroll=False)

        pltpu.sync_copy(table_hbm.at[pl.ds(srow + _SCH * _CHUNK, _STAIL)],
                        rin[0].at[pl.ds(0, _STAIL)])
        accs = scan_rows(0, _STAIL, accs)
        m = accs[0]
        for a in accs[1:]:
            m = jnp.maximum(m, a)

        red_v[...] = m
        pltpu.sync_copy(red_v, shared.at[sid])
        plsc.subcore_barrier()
        pltpu.sync_copy(shared, redall_v)
        for t in range(16):
            m = jnp.maximum(m, redall_v[t, :])
        absmax_v = jnp.full((16,), jnp.max(m), jnp.float32)
        s = jnp.maximum(absmax_v / QMAX, 1e-8)
        rs = 1.0 / s

        def dequant(b):
            def row_body(rr, _):
                for u in range(2):
                    for j in range(2):
                        for c in range(DIM // 16):
                            v = rin[b][4 * rr + 2 * u + j,
                                       pl.ds(c * 16, 16)]
                            t = v * rs
                            tb = plsc.bitcast(t, jnp.uint32)
                            csign = plsc.bitcast(
                                (tb & _SIGN_MASK) | _MAGIC_BITS, jnp.float32)
                            q = (t + csign) - csign
                            rout[b][2 * rr + u,
                                    pl.ds(j * DIM + c * 16, 16)] = q * s
                return 0

            lax.fori_loop(0, _CHUNK // 4, row_body, 0, unroll=False)

        def start_gather(b, j):
            off = base + j * _CHUNK
            pltpu.sync_copy(idx_hbm.at[pl.ds(off, _CHUNK)], idx_v.at[b])
            pltpu.async_copy(table_hbm.at[idx_v.at[b]], rin[b], sem_g[b])

        for b in range(2):
            start_gather(b, b)

        def pair_body(p, _):
            for b in range(2):
                j = 2 * p + b
                off2 = base2 + j * (_CHUNK // 2)
                pltpu.make_async_copy(table_hbm.at[idx_v.at[b]], rin[b],
                                      sem_g[b]).wait()
                dequant(b)
                pltpu.async_copy(rout[b],
                                 out_hbm.at[pl.ds(off2, _CHUNK // 2)],
                                 sem_o[b])

                @pl.when(p < _NCHUNK // 2 - 1)
                def _prefetch():
                    pltpu.make_async_copy(
                        rout[b], out_hbm.at[pl.ds(base2, _CHUNK // 2)],
                        sem_o[b]).wait()
                    start_gather(b, j + 2)

            return 0

        lax.fori_loop(0, _NCHUNK // 2, pair_body, 0, unroll=False)

        for b in range(2):
            pltpu.make_async_copy(rout[b],
                                  out_hbm.at[pl.ds(base2, _CHUNK // 2)],
                                  sem_o[b]).wait()

    return k(table, idx_flat)


def kernel(x, weight):
    out = _gather_fq(weight, x.reshape(-1))
    return out.reshape(BATCH, HIST, DIM)
